# baseline (speedup 1.0000x reference)
import jax
import jax.numpy as jnp
from jax import lax
from jax.experimental import pallas as pl
from jax.experimental.pallas import tpu as pltpu

_B = 16
_D = 64
_HW = 1024
_N_TOK = _B * _HW


def _fill_body(emb_ref, enc_ref, q_ref):
    for c in range(_D):
        v = emb_ref[c]
        q_ref[:, c : c + 1, :, :] = jnp.full((_B, 1, 32, 32), v, jnp.float32)
    enc_ref[...] = jnp.full((128, 128), 1.0, jnp.float32)


def kernel(x, embeddings):
    del x
    emb_flat = embeddings.reshape(_D)
    enc2, quantized = pl.pallas_call(
        _fill_body,
        in_specs=[pl.BlockSpec(memory_space=pltpu.SMEM)],
        out_shape=[
            jax.ShapeDtypeStruct((128, 128), jnp.float32),
            jax.ShapeDtypeStruct((_B, _D, 32, 32), jnp.float32),
        ],
    )(emb_flat)
    encodings = enc2.reshape(_N_TOK, 1)
    return (encodings, quantized)


# speedup vs baseline: 1.8902x; 1.8902x over previous
import jax
import jax.numpy as jnp
from jax import lax
from jax.experimental import pallas as pl

_B = 16
_D = 64
_HW = 1024
_N_TOK = _B * _HW


def _fill_body(emb_ref, q_ref, enc_ref):
    i = pl.program_id(0)
    col = emb_ref[...]
    q_ref[...] = lax.broadcast_in_dim(col, (_B // 2, _D, _HW), (1, 2))

    @pl.when(i == 0)
    def _():
        enc_ref[...] = jnp.full((128, 128), 1.0, jnp.float32)


def kernel(x, embeddings):
    del x
    emb_col = embeddings.reshape(_D, 1)
    q3, enc2 = pl.pallas_call(
        _fill_body,
        grid=(2,),
        in_specs=[pl.BlockSpec((_D, 1), lambda i: (0, 0))],
        out_specs=[
            pl.BlockSpec((_B // 2, _D, _HW), lambda i: (i, 0, 0)),
            pl.BlockSpec((128, 128), lambda i: (0, 0)),
        ],
        out_shape=[
            jax.ShapeDtypeStruct((_B, _D, _HW), jnp.float32),
            jax.ShapeDtypeStruct((128, 128), jnp.float32),
        ],
    )(emb_col)
    quantized = q3.reshape(_B, _D, 32, 32)
    encodings = enc2.reshape(_N_TOK, 1)
    return (encodings, quantized)
